# baseline (speedup 1.0000x reference)
import functools

import jax
import jax.numpy as jnp
from jax import lax
from jax.experimental import pallas as pl
from jax.experimental.pallas import tpu as pltpu
from jax.experimental.pallas import tpu_sc as plsc

_NC = 2
_NS = 16
_NW = _NC * _NS
_CHK = 80
_RPW = 4
C = 16


def _sc_gather_chunk(table, src_s, idx4, *, E):
    NP = table.shape[0]
    D = table.shape[1]
    NROW = NP // _CHK

    mesh = plsc.VectorSubcoreMesh(core_axis_name="c", subcore_axis_name="s")

    @functools.partial(
        pl.kernel,
        mesh=mesh,
        out_type=jax.ShapeDtypeStruct((C, NROW, _CHK, D), jnp.float32),
        scratch_types=[
            pltpu.VMEM((C, _RPW, _CHK), jnp.int32),
            pltpu.VMEM((C, _RPW, _CHK), jnp.int32),
            pltpu.VMEM((_RPW, _CHK, D), jnp.float32),
            pltpu.VMEM((_RPW, _CHK, D), jnp.float32),
            pltpu.SemaphoreType.DMA,
            pltpu.SemaphoreType.DMA,
            pltpu.SemaphoreType.DMA,
        ],
    )
    def k(table_hbm, srcs_hbm, idx_hbm, out_hbm, idx_v, sid_v, r0, r1,
          sem_s, sem_r, sem_o):
        wid = lax.axis_index("s") * _NC + lax.axis_index("c")
        row0 = wid * _RPW
        bufs = (r0, r1)
        pltpu.sync_copy(idx_hbm.at[wid], idx_v)
        sid_h = [[pltpu.async_copy(srcs_hbm.at[idx_v.at[c, j]],
                                   sid_v.at[c, j], sem_s)
                  for j in range(_RPW)] for c in range(C)]

        def fire_rows(c):
            for h in sid_h[c]:
                h.wait()
            buf = bufs[c % 2]
            return [pltpu.async_copy(table_hbm.at[sid_v.at[c, j]],
                                     buf.at[j], sem_r)
                    for j in range(_RPW)]

        row_h = fire_rows(0)
        out_h = [None] * C
        for c in range(C):
            for h in row_h:
                h.wait()
            out_h[c] = pltpu.async_copy(
                bufs[c % 2], out_hbm.at[c, pl.ds(row0, _RPW)], sem_o)
            if c + 1 < C:
                if c >= 1:
                    out_h[c - 1].wait()
                row_h = fire_rows(c + 1)
        if C >= 2:
            out_h[C - 2].wait()
        out_h[C - 1].wait()

    return k(table, src_s, idx4)


def _tc_reduce(Xc, cnt_rel, acc, *, mode):
    NP, D = acc.shape
    blk = 1024

    def body(xc_ref, cnt_ref, acc_ref, out_ref):
        a = acc_ref[...]
        cnt = cnt_ref[...]
        for s in range(C):
            x = xc_ref[s]
            valid = cnt > s
            if mode == "sum":
                a = a + jnp.where(valid, x, 0.0)
            else:
                a = jnp.maximum(a, jnp.where(valid, x, -jnp.inf))
        out_ref[...] = a

    return pl.pallas_call(
        body,
        grid=(NP // blk,),
        in_specs=[
            pl.BlockSpec((C, blk, D), lambda i: (0, i, 0)),
            pl.BlockSpec((blk, 1), lambda i: (i, 0)),
            pl.BlockSpec((blk, D), lambda i: (i, 0)),
        ],
        out_specs=pl.BlockSpec((blk, D), lambda i: (i, 0)),
        out_shape=jax.ShapeDtypeStruct((NP, D), jnp.float32),
        input_output_aliases={2: 0},
    )(Xc, cnt_rel, acc)


def _tc_lstm_chunk(md_rel, Xc, cnt_rel, h, c, W_ih, W_hh, bias):
    NP, D = h.shape
    blk = 512

    def body(md_ref, xc_ref, cnt_ref, h_ref, c_ref, wi_ref, wh_ref, b_ref,
             ho_ref, co_ref):
        hh = h_ref[...]
        cc = c_ref[...]
        cnt = cnt_ref[...]
        wi = wi_ref[...]
        wh = wh_ref[...]
        b = b_ref[...]
        md = md_ref[0]
        for s in range(C):
            x = jnp.where(cnt > s, xc_ref[s], 0.0)
            g = (jnp.dot(x, wi, preferred_element_type=jnp.float32)
                 + jnp.dot(hh, wh, preferred_element_type=jnp.float32) + b)
            gi = jax.nn.sigmoid(g[:, 0 * D:1 * D])
            gf = jax.nn.sigmoid(g[:, 1 * D:2 * D])
            gg = jnp.tanh(g[:, 2 * D:3 * D])
            go = jax.nn.sigmoid(g[:, 3 * D:4 * D])
            cn = gf * cc + gi * gg
            hn = go * jnp.tanh(cn)
            upd = s < md
            hh = jnp.where(upd, hn, hh)
            cc = jnp.where(upd, cn, cc)
        ho_ref[...] = hh
        co_ref[...] = cc

    return pl.pallas_call(
        body,
        grid=(NP // blk,),
        in_specs=[
            pl.BlockSpec(memory_space=pltpu.SMEM),
            pl.BlockSpec((C, blk, D), lambda i: (0, i, 0)),
            pl.BlockSpec((blk, 1), lambda i: (i, 0)),
            pl.BlockSpec((blk, D), lambda i: (i, 0)),
            pl.BlockSpec((blk, D), lambda i: (i, 0)),
            pl.BlockSpec((D, 4 * D), lambda i: (0, 0)),
            pl.BlockSpec((D, 4 * D), lambda i: (0, 0)),
            pl.BlockSpec((1, 4 * D), lambda i: (0, 0)),
        ],
        out_specs=[
            pl.BlockSpec((blk, D), lambda i: (i, 0)),
            pl.BlockSpec((blk, D), lambda i: (i, 0)),
        ],
        out_shape=[
            jax.ShapeDtypeStruct((NP, D), jnp.float32),
            jax.ShapeDtypeStruct((NP, D), jnp.float32),
        ],
        input_output_aliases={3: 0, 4: 1},
    )(md_rel, Xc, cnt_rel, h, c, W_ih, W_hh, bias)


def _tc_combine(agg, hprev, Wl, Wr, b, cnt, *, mode):
    NP, D = agg.shape
    blk = 512

    def body(agg_ref, hp_ref, wl_ref, wr_ref, b_ref, cnt_ref, out_ref):
        a = agg_ref[...]
        cntf = cnt_ref[...].astype(jnp.float32)
        if mode == "mean":
            a = a / jnp.maximum(cntf, 1.0)
        elif mode == "max":
            a = jnp.where(cntf > 0.0, a, 0.0)
        o = (jnp.dot(a, wl_ref[...], preferred_element_type=jnp.float32)
             + jnp.dot(hp_ref[...], wr_ref[...],
                       preferred_element_type=jnp.float32)
             + b_ref[...])
        if mode != "plain":
            o = jnp.maximum(o, 0.0)
        out_ref[...] = o

    return pl.pallas_call(
        body,
        grid=(NP // blk,),
        in_specs=[
            pl.BlockSpec((blk, D), lambda i: (i, 0)),
            pl.BlockSpec((blk, D), lambda i: (i, 0)),
            pl.BlockSpec((D, D), lambda i: (0, 0)),
            pl.BlockSpec((D, D), lambda i: (0, 0)),
            pl.BlockSpec((1, D), lambda i: (0, 0)),
            pl.BlockSpec((blk, 1), lambda i: (i, 0)),
        ],
        out_specs=pl.BlockSpec((blk, D), lambda i: (i, 0)),
        out_shape=jax.ShapeDtypeStruct((NP, D), jnp.float32),
    )(agg, hprev, Wl, Wr, b, cnt)


def _tc_jk(h1, h2, h3, Wf_ih, Wf_hh, bf, Wb_ih, Wb_hh, bb, watt, *, H):
    NP, D = h1.shape
    blk = 512

    def body(h1_ref, h2_ref, h3_ref, wfi_ref, wfh_ref, bf_ref,
             wbi_ref, wbh_ref, bb_ref, wa_ref, out_ref):
        x1 = h1_ref[...]
        x2 = h2_ref[...]
        x3 = h3_ref[...]
        seq = (x1, x2, x3)

        def cell(x, h, c, wi, wh, b):
            g = (jnp.dot(x, wi, preferred_element_type=jnp.float32)
                 + jnp.dot(h, wh, preferred_element_type=jnp.float32) + b)
            gi = jax.nn.sigmoid(g[:, 0 * H:1 * H])
            gf = jax.nn.sigmoid(g[:, 1 * H:2 * H])
            gg = jnp.tanh(g[:, 2 * H:3 * H])
            go = jax.nn.sigmoid(g[:, 3 * H:4 * H])
            c2 = gf * c + gi * gg
            return go * jnp.tanh(c2), c2

        wfi = wfi_ref[...]
        wfh = wfh_ref[...]
        bfv = bf_ref[...]
        wbi = wbi_ref[...]
        wbh = wbh_ref[...]
        bbv = bb_ref[...]
        z = jnp.zeros((x1.shape[0], H), jnp.float32)
        hf, cf = z, z
        hs_f = []
        for t in range(3):
            hf, cf = cell(seq[t], hf, cf, wfi, wfh, bfv)
            hs_f.append(hf)
        hb, cb = z, z
        hs_b = [None, None, None]
        for k in range(3):
            t = 2 - k
            hb, cb = cell(seq[t], hb, cb, wbi, wbh, bbv)
            hs_b[t] = hb
        wa = wa_ref[...]
        wa_f = wa[:, :H]
        wa_b = wa[:, H:]
        atts = []
        for t in range(3):
            att = (jnp.sum(hs_f[t] * wa_f, axis=1, keepdims=True)
                   + jnp.sum(hs_b[t] * wa_b, axis=1, keepdims=True))
            atts.append(att)
        m = jnp.maximum(atts[0], jnp.maximum(atts[1], atts[2]))
        e0 = jnp.exp(atts[0] - m)
        e1 = jnp.exp(atts[1] - m)
        e2 = jnp.exp(atts[2] - m)
        z_sum = e0 + e1 + e2
        out_ref[...] = (e0 * x1 + e1 * x2 + e2 * x3) / z_sum

    return pl.pallas_call(
        body,
        grid=(NP // blk,),
        in_specs=[
            pl.BlockSpec((blk, D), lambda i: (i, 0)),
            pl.BlockSpec((blk, D), lambda i: (i, 0)),
            pl.BlockSpec((blk, D), lambda i: (i, 0)),
            pl.BlockSpec((D, 4 * H), lambda i: (0, 0)),
            pl.BlockSpec((H, 4 * H), lambda i: (0, 0)),
            pl.BlockSpec((1, 4 * H), lambda i: (0, 0)),
            pl.BlockSpec((D, 4 * H), lambda i: (0, 0)),
            pl.BlockSpec((H, 4 * H), lambda i: (0, 0)),
            pl.BlockSpec((1, 4 * H), lambda i: (0, 0)),
            pl.BlockSpec((1, 2 * H), lambda i: (0, 0)),
        ],
        out_specs=pl.BlockSpec((blk, D), lambda i: (i, 0)),
        out_shape=jax.ShapeDtypeStruct((NP, D), jnp.float32),
    )(h1, h2, h3, Wf_ih, Wf_hh, bf, Wb_ih, Wb_hh, bb, watt)


def _make_idx(starts_p, counts_p, t0, *, E, NP):
    offs = t0 + jnp.arange(C, dtype=jnp.int32)
    valid = offs[:, None] < counts_p[None, :]
    idx = jnp.where(valid, starts_p[None, :] + offs[:, None], E - 1)
    return idx.reshape(C, _NW, _RPW, _CHK).transpose(1, 0, 2, 3)


def _agg_pass(table, src_s, starts_p, counts_p, max_deg, *, mode, E):
    NP, D = table.shape
    if mode == "sum":
        init = jnp.zeros((NP, D), jnp.float32)
    else:
        init = jnp.full((NP, D), -jnp.inf, jnp.float32)
    K = (max_deg + C - 1) // C

    def cond(st):
        return st[0] < K

    def body(st):
        i, acc = st
        t0 = i * C
        idx4 = _make_idx(starts_p, counts_p, t0, E=E, NP=NP)
        Xc = _sc_gather_chunk(table, src_s, idx4, E=E).reshape(C, NP, D)
        cnt_rel = (counts_p - t0)[:, None]
        acc = _tc_reduce(Xc, cnt_rel, acc, mode=mode)
        return (i + jnp.int32(1), acc)

    _, acc = lax.while_loop(cond, body, (jnp.int32(0), init))
    return acc


def _lstm_pass(table, src_s, starts_p, counts_p, max_deg, W_ih, W_hh, bias,
               *, E):
    NP, D = table.shape
    K = (max_deg + C - 1) // C
    h0 = jnp.zeros((NP, D), jnp.float32)
    c0 = jnp.zeros((NP, D), jnp.float32)

    def cond(st):
        return st[0] < K

    def body(st):
        i, h, c = st
        t0 = i * C
        idx4 = _make_idx(starts_p, counts_p, t0, E=E, NP=NP)
        Xc = _sc_gather_chunk(table, src_s, idx4, E=E).reshape(C, NP, D)
        cnt_rel = (counts_p - t0)[:, None]
        md_rel = jnp.reshape(max_deg - t0, (1,)).astype(jnp.int32)
        h, c = _tc_lstm_chunk(md_rel, Xc, cnt_rel, h, c, W_ih, W_hh, bias)
        return (i + jnp.int32(1), h, c)

    _, h, _ = lax.while_loop(cond, body, (jnp.int32(0), h0, c0))
    return h


def kernel(x, edge_index, W_l1, W_r1, b1, W_l2, W_r2, b2, W_ih3, W_hh3, b_ih3,
           b_hh3, W_l3, W_r3, b3, Wf_ih, Wf_hh, bf_ih, bf_hh, Wb_ih, Wb_hh,
           bb_ih, bb_hh, W_att, b_att):
    N, D = x.shape
    E = edge_index.shape[1]
    H = Wf_hh.shape[0]
    GR = _NW * _RPW * _CHK
    NP = ((N + GR - 1) // GR) * GR

    src = edge_index[0].astype(jnp.int32)
    dst = edge_index[1].astype(jnp.int32)
    perm = jnp.argsort(dst, stable=True)
    src_s = src[perm]
    counts = jnp.bincount(dst, length=N).astype(jnp.int32)
    starts = jnp.concatenate(
        [jnp.zeros((1,), jnp.int32), jnp.cumsum(counts)[:-1].astype(jnp.int32)])
    max_deg = jnp.max(counts)

    x_p = jnp.zeros((NP, D), jnp.float32).at[:N].set(x)
    counts_p = jnp.zeros((NP,), jnp.int32).at[:N].set(counts)
    starts_p = jnp.zeros((NP,), jnp.int32).at[:N].set(starts)
    cnt_col = counts_p[:, None]

    b1r = b1.reshape(1, D)
    b2r = b2.reshape(1, D)
    b3r = b3.reshape(1, D)
    b3s = (b_ih3 + b_hh3).reshape(1, 4 * D)
    bfr = (bf_ih + bf_hh).reshape(1, 4 * H)
    bbr = (bb_ih + bb_hh).reshape(1, 4 * H)
    watt = W_att.reshape(1, 2 * H)

    agg1 = _agg_pass(x_p, src_s, starts_p, counts_p, max_deg, mode="sum", E=E)
    h1 = _tc_combine(agg1, x_p, W_l1, W_r1, b1r, cnt_col, mode="mean")
    agg2 = _agg_pass(h1, src_s, starts_p, counts_p, max_deg, mode="max", E=E)
    h2 = _tc_combine(agg2, h1, W_l2, W_r2, b2r, cnt_col, mode="max")
    m3 = _lstm_pass(h2, src_s, starts_p, counts_p, max_deg, W_ih3, W_hh3, b3s,
                    E=E)
    h3 = _tc_combine(m3, h2, W_l3, W_r3, b3r, cnt_col, mode="plain")
    hout = _tc_jk(h1, h2, h3, Wf_ih, Wf_hh, bfr, Wb_ih, Wb_hh, bbr, watt, H=H)
    h = hout[:N]
    return (h, h)


# speedup vs baseline: 14.0589x; 14.0589x over previous
import functools

import jax
import jax.numpy as jnp
from jax import lax
from jax.experimental import pallas as pl
from jax.experimental.pallas import tpu as pltpu
from jax.experimental.pallas import tpu_sc as plsc

_NC = 2
_NS = 16
_NW = _NC * _NS
_CHK = 80
_RPW = 4
C = 8


def _sc_gather_chunk_w(table, src_slot, idx4, *, W):
    NP = table.shape[0]
    D = table.shape[1]
    NROW = NP // _CHK

    mesh = plsc.VectorSubcoreMesh(core_axis_name="c", subcore_axis_name="s")

    @functools.partial(
        pl.kernel,
        mesh=mesh,
        out_type=jax.ShapeDtypeStruct((C, NROW, _CHK, D), jnp.float32),
        scratch_types=[
            pltpu.VMEM((C, _RPW, _CHK), jnp.int32),
            pltpu.VMEM((C, _RPW, _CHK), jnp.int32),
            pltpu.VMEM((W, _CHK, D), jnp.float32),
            pltpu.VMEM((W, _CHK, D), jnp.float32),
            pltpu.SemaphoreType.DMA,
            pltpu.SemaphoreType.DMA,
            pltpu.SemaphoreType.DMA,
        ],
    )
    def k(table_hbm, srcs_hbm, idx_hbm, out_hbm, idx_v, sid_v, r0, r1,
          sem_s, sem_r, sem_o):
        wid = lax.axis_index("s") * _NC + lax.axis_index("c")
        row0 = wid * _RPW
        bufs = (r0, r1)
        pltpu.sync_copy(idx_hbm.at[wid], idx_v)
        sid_h = [[pltpu.async_copy(srcs_hbm.at[idx_v.at[c, j]],
                                   sid_v.at[c, j], sem_s)
                  for j in range(W)] for c in range(C)]

        def fire_rows(c):
            for h in sid_h[c]:
                h.wait()
            buf = bufs[c % 2]
            return [pltpu.async_copy(table_hbm.at[sid_v.at[c, j]],
                                     buf.at[j], sem_r)
                    for j in range(W)]

        row_h = fire_rows(0)
        out_h = [None] * C
        for c in range(C):
            for h in row_h:
                h.wait()
            out_h[c] = pltpu.async_copy(
                bufs[c % 2], out_hbm.at[c, pl.ds(row0, W)], sem_o)
            if c + 1 < C:
                if c >= 1:
                    out_h[c - 1].wait()
                row_h = fire_rows(c + 1)
        if C >= 2:
            out_h[C - 2].wait()
        out_h[C - 1].wait()

    return k(table, src_slot, idx4)


def _sc_gather_chunk(table, src_slot, idx4, nch0, *, E):
    del E
    branches = [functools.partial(_sc_gather_chunk_w, W=w)
                for w in range(1, _RPW + 1)]
    sel = jnp.clip(nch0, 1, _RPW) - 1
    return lax.switch(sel, branches, table, src_slot, idx4)


def _sc_permute(table, perm3):
    NP = table.shape[0]
    D = table.shape[1]
    NROW = NP // _CHK
    RP = NROW // _NW

    mesh = plsc.VectorSubcoreMesh(core_axis_name="c", subcore_axis_name="s")

    @functools.partial(
        pl.kernel,
        mesh=mesh,
        out_type=jax.ShapeDtypeStruct((NROW, _CHK, D), jnp.float32),
        scratch_types=[
            pltpu.VMEM((RP, _CHK), jnp.int32),
            pltpu.VMEM((RP, _CHK, D), jnp.float32),
            pltpu.SemaphoreType.DMA,
        ],
    )
    def k(table_hbm, perm_hbm, out_hbm, idx_v, rows_v, sem):
        wid = lax.axis_index("s") * _NC + lax.axis_index("c")
        row0 = wid * RP
        pltpu.sync_copy(perm_hbm.at[wid], idx_v)
        hs = [pltpu.async_copy(table_hbm.at[idx_v.at[j]], rows_v.at[j], sem)
              for j in range(RP)]
        for h in hs:
            h.wait()
        pltpu.sync_copy(rows_v, out_hbm.at[pl.ds(row0, RP)])

    return k(table, perm3)


def _tc_reduce(Xc, cnt_rel, acc, *, mode):
    NP, D = acc.shape
    blk = 1024

    def body(xc_ref, cnt_ref, acc_ref, out_ref):
        a = acc_ref[...]
        cnt = cnt_ref[...]
        for s in range(C):
            x = xc_ref[s]
            valid = cnt > s
            if mode == "sum":
                a = a + jnp.where(valid, x, 0.0)
            else:
                a = jnp.maximum(a, jnp.where(valid, x, -jnp.inf))
        out_ref[...] = a

    return pl.pallas_call(
        body,
        grid=(NP // blk,),
        in_specs=[
            pl.BlockSpec((C, blk, D), lambda i: (0, i, 0)),
            pl.BlockSpec((blk, 1), lambda i: (i, 0)),
            pl.BlockSpec((blk, D), lambda i: (i, 0)),
        ],
        out_specs=pl.BlockSpec((blk, D), lambda i: (i, 0)),
        out_shape=jax.ShapeDtypeStruct((NP, D), jnp.float32),
        input_output_aliases={2: 0},
    )(Xc, cnt_rel, acc)


def _tc_lstm_chunk(md_rel, Xc, cnt_rel, h, c, W_ih, W_hh, bias):
    NP, D = h.shape
    blk = 512

    def body(md_ref, xc_ref, cnt_ref, h_ref, c_ref, wi_ref, wh_ref, b_ref,
             ho_ref, co_ref):
        hh = h_ref[...]
        cc = c_ref[...]
        cnt = cnt_ref[...]
        wi = wi_ref[...]
        wh = wh_ref[...]
        b = b_ref[...]
        md = md_ref[0]
        for s in range(C):
            x = jnp.where(cnt > s, xc_ref[s], 0.0)
            g = (jnp.dot(x, wi, preferred_element_type=jnp.float32)
                 + jnp.dot(hh, wh, preferred_element_type=jnp.float32) + b)
            gi = jax.nn.sigmoid(g[:, 0 * D:1 * D])
            gf = jax.nn.sigmoid(g[:, 1 * D:2 * D])
            gg = jnp.tanh(g[:, 2 * D:3 * D])
            go = jax.nn.sigmoid(g[:, 3 * D:4 * D])
            cn = gf * cc + gi * gg
            hn = go * jnp.tanh(cn)
            upd = s < md
            hh = jnp.where(upd, hn, hh)
            cc = jnp.where(upd, cn, cc)
        ho_ref[...] = hh
        co_ref[...] = cc

    return pl.pallas_call(
        body,
        grid=(NP // blk,),
        in_specs=[
            pl.BlockSpec(memory_space=pltpu.SMEM),
            pl.BlockSpec((C, blk, D), lambda i: (0, i, 0)),
            pl.BlockSpec((blk, 1), lambda i: (i, 0)),
            pl.BlockSpec((blk, D), lambda i: (i, 0)),
            pl.BlockSpec((blk, D), lambda i: (i, 0)),
            pl.BlockSpec((D, 4 * D), lambda i: (0, 0)),
            pl.BlockSpec((D, 4 * D), lambda i: (0, 0)),
            pl.BlockSpec((1, 4 * D), lambda i: (0, 0)),
        ],
        out_specs=[
            pl.BlockSpec((blk, D), lambda i: (i, 0)),
            pl.BlockSpec((blk, D), lambda i: (i, 0)),
        ],
        out_shape=[
            jax.ShapeDtypeStruct((NP, D), jnp.float32),
            jax.ShapeDtypeStruct((NP, D), jnp.float32),
        ],
        input_output_aliases={3: 0, 4: 1},
    )(md_rel, Xc, cnt_rel, h, c, W_ih, W_hh, bias)


def _tc_combine(agg, hprev, Wl, Wr, b, cnt, *, mode):
    NP, D = agg.shape
    blk = 512

    def body(agg_ref, hp_ref, wl_ref, wr_ref, b_ref, cnt_ref, out_ref):
        a = agg_ref[...]
        cntf = cnt_ref[...].astype(jnp.float32)
        if mode == "mean":
            a = a / jnp.maximum(cntf, 1.0)
        elif mode == "max":
            a = jnp.where(cntf > 0.0, a, 0.0)
        o = (jnp.dot(a, wl_ref[...], preferred_element_type=jnp.float32)
             + jnp.dot(hp_ref[...], wr_ref[...],
                       preferred_element_type=jnp.float32)
             + b_ref[...])
        if mode != "plain":
            o = jnp.maximum(o, 0.0)
        out_ref[...] = o

    return pl.pallas_call(
        body,
        grid=(NP // blk,),
        in_specs=[
            pl.BlockSpec((blk, D), lambda i: (i, 0)),
            pl.BlockSpec((blk, D), lambda i: (i, 0)),
            pl.BlockSpec((D, D), lambda i: (0, 0)),
            pl.BlockSpec((D, D), lambda i: (0, 0)),
            pl.BlockSpec((1, D), lambda i: (0, 0)),
            pl.BlockSpec((blk, 1), lambda i: (i, 0)),
        ],
        out_specs=pl.BlockSpec((blk, D), lambda i: (i, 0)),
        out_shape=jax.ShapeDtypeStruct((NP, D), jnp.float32),
    )(agg, hprev, Wl, Wr, b, cnt)


def _tc_jk(h1, h2, h3, Wf_ih, Wf_hh, bf, Wb_ih, Wb_hh, bb, watt, *, H):
    NP, D = h1.shape
    blk = 512

    def body(h1_ref, h2_ref, h3_ref, wfi_ref, wfh_ref, bf_ref,
             wbi_ref, wbh_ref, bb_ref, wa_ref, out_ref):
        x1 = h1_ref[...]
        x2 = h2_ref[...]
        x3 = h3_ref[...]
        seq = (x1, x2, x3)

        def cell(x, h, c, wi, wh, b):
            g = (jnp.dot(x, wi, preferred_element_type=jnp.float32)
                 + jnp.dot(h, wh, preferred_element_type=jnp.float32) + b)
            gi = jax.nn.sigmoid(g[:, 0 * H:1 * H])
            gf = jax.nn.sigmoid(g[:, 1 * H:2 * H])
            gg = jnp.tanh(g[:, 2 * H:3 * H])
            go = jax.nn.sigmoid(g[:, 3 * H:4 * H])
            c2 = gf * c + gi * gg
            return go * jnp.tanh(c2), c2

        wfi = wfi_ref[...]
        wfh = wfh_ref[...]
        bfv = bf_ref[...]
        wbi = wbi_ref[...]
        wbh = wbh_ref[...]
        bbv = bb_ref[...]
        z = jnp.zeros((x1.shape[0], H), jnp.float32)
        hf, cf = z, z
        hs_f = []
        for t in range(3):
            hf, cf = cell(seq[t], hf, cf, wfi, wfh, bfv)
            hs_f.append(hf)
        hb, cb = z, z
        hs_b = [None, None, None]
        for k in range(3):
            t = 2 - k
            hb, cb = cell(seq[t], hb, cb, wbi, wbh, bbv)
            hs_b[t] = hb
        wa = wa_ref[...]
        wa_f = wa[:, :H]
        wa_b = wa[:, H:]
        atts = []
        for t in range(3):
            att = (jnp.sum(hs_f[t] * wa_f, axis=1, keepdims=True)
                   + jnp.sum(hs_b[t] * wa_b, axis=1, keepdims=True))
            atts.append(att)
        m = jnp.maximum(atts[0], jnp.maximum(atts[1], atts[2]))
        e0 = jnp.exp(atts[0] - m)
        e1 = jnp.exp(atts[1] - m)
        e2 = jnp.exp(atts[2] - m)
        z_sum = e0 + e1 + e2
        out_ref[...] = (e0 * x1 + e1 * x2 + e2 * x3) / z_sum

    return pl.pallas_call(
        body,
        grid=(NP // blk,),
        in_specs=[
            pl.BlockSpec((blk, D), lambda i: (i, 0)),
            pl.BlockSpec((blk, D), lambda i: (i, 0)),
            pl.BlockSpec((blk, D), lambda i: (i, 0)),
            pl.BlockSpec((D, 4 * H), lambda i: (0, 0)),
            pl.BlockSpec((H, 4 * H), lambda i: (0, 0)),
            pl.BlockSpec((1, 4 * H), lambda i: (0, 0)),
            pl.BlockSpec((D, 4 * H), lambda i: (0, 0)),
            pl.BlockSpec((H, 4 * H), lambda i: (0, 0)),
            pl.BlockSpec((1, 4 * H), lambda i: (0, 0)),
            pl.BlockSpec((1, 2 * H), lambda i: (0, 0)),
        ],
        out_specs=pl.BlockSpec((blk, D), lambda i: (i, 0)),
        out_shape=jax.ShapeDtypeStruct((NP, D), jnp.float32),
    )(h1, h2, h3, Wf_ih, Wf_hh, bf, Wb_ih, Wb_hh, bb, watt)


def _chunk_meta(starts_s, counts_s, t0, *, E):
    offs = t0 + jnp.arange(C, dtype=jnp.int32)
    idx = jnp.minimum(starts_s[None, :] + offs[:, None], E - 1)
    idx4 = idx.reshape(C, _NW, _RPW, _CHK).transpose(1, 0, 2, 3)
    act = jnp.sum(counts_s > t0)
    per_w = (act + _NW - 1) // _NW
    nch0 = ((per_w + _CHK - 1) // _CHK).astype(jnp.int32)
    return idx4, nch0


def _agg_pass(table, src_slot, starts_s, counts_s, max_deg, *, mode, E):
    NP, D = table.shape
    if mode == "sum":
        init = jnp.zeros((NP, D), jnp.float32)
    else:
        init = jnp.full((NP, D), -jnp.inf, jnp.float32)
    K = (max_deg + C - 1) // C

    def cond(st):
        return st[0] < K

    def body(st):
        i, acc = st
        t0 = i * C
        idx4, nch0 = _chunk_meta(starts_s, counts_s, t0, E=E)
        Xc = _sc_gather_chunk(table, src_slot, idx4, nch0,
                              E=E).reshape(C, NP, D)
        cnt_rel = (counts_s - t0)[:, None]
        acc = _tc_reduce(Xc, cnt_rel, acc, mode=mode)
        return (i + jnp.int32(1), acc)

    _, acc = lax.while_loop(cond, body, (jnp.int32(0), init))
    return acc


def _lstm_pass(table, src_slot, starts_s, counts_s, max_deg, W_ih, W_hh, bias,
               *, E):
    NP, D = table.shape
    K = (max_deg + C - 1) // C
    h0 = jnp.zeros((NP, D), jnp.float32)
    c0 = jnp.zeros((NP, D), jnp.float32)

    def cond(st):
        return st[0] < K

    def body(st):
        i, h, c = st
        t0 = i * C
        idx4, nch0 = _chunk_meta(starts_s, counts_s, t0, E=E)
        Xc = _sc_gather_chunk(table, src_slot, idx4, nch0,
                              E=E).reshape(C, NP, D)
        cnt_rel = (counts_s - t0)[:, None]
        md_rel = jnp.reshape(max_deg - t0, (1,)).astype(jnp.int32)
        h, c = _tc_lstm_chunk(md_rel, Xc, cnt_rel, h, c, W_ih, W_hh, bias)
        return (i + jnp.int32(1), h, c)

    _, h, _ = lax.while_loop(cond, body, (jnp.int32(0), h0, c0))
    return h


def kernel(x, edge_index, W_l1, W_r1, b1, W_l2, W_r2, b2, W_ih3, W_hh3, b_ih3,
           b_hh3, W_l3, W_r3, b3, Wf_ih, Wf_hh, bf_ih, bf_hh, Wb_ih, Wb_hh,
           bb_ih, bb_hh, W_att, b_att):
    N, D = x.shape
    E = edge_index.shape[1]
    H = Wf_hh.shape[0]
    GR = _NW * _RPW * _CHK
    NP = ((N + GR - 1) // GR) * GR

    src = edge_index[0].astype(jnp.int32)
    dst = edge_index[1].astype(jnp.int32)
    perm = jnp.argsort(dst, stable=True)
    src_s = src[perm]
    counts = jnp.bincount(dst, length=N).astype(jnp.int32)
    starts = jnp.concatenate(
        [jnp.zeros((1,), jnp.int32), jnp.cumsum(counts)[:-1].astype(jnp.int32)])
    max_deg = jnp.max(counts)

    x_p = jnp.zeros((NP, D), jnp.float32).at[:N].set(x)
    counts_p = jnp.zeros((NP,), jnp.int32).at[:N].set(counts)
    starts_p = jnp.zeros((NP,), jnp.int32).at[:N].set(starts)

    SPW = NP // _NW
    RP = SPW // _CHK
    nperm = jnp.argsort(-counts_p).astype(jnp.int32)
    slotperm = nperm.reshape(SPW, _NW).transpose(1, 0).reshape(NP)
    slot_of = jnp.zeros((NP,), jnp.int32).at[slotperm].set(
        jnp.arange(NP, dtype=jnp.int32))
    counts_s = counts_p[slotperm]
    starts_s = starts_p[slotperm]
    src_slot = slot_of[src_s]
    x_s = _sc_permute(x_p, slotperm.reshape(_NW, RP, _CHK)).reshape(NP, D)
    cnt_col = counts_s[:, None]

    b1r = b1.reshape(1, D)
    b2r = b2.reshape(1, D)
    b3r = b3.reshape(1, D)
    b3s = (b_ih3 + b_hh3).reshape(1, 4 * D)
    bfr = (bf_ih + bf_hh).reshape(1, 4 * H)
    bbr = (bb_ih + bb_hh).reshape(1, 4 * H)
    watt = W_att.reshape(1, 2 * H)

    agg1 = _agg_pass(x_s, src_slot, starts_s, counts_s, max_deg, mode="sum",
                     E=E)
    h1 = _tc_combine(agg1, x_s, W_l1, W_r1, b1r, cnt_col, mode="mean")
    agg2 = _agg_pass(h1, src_slot, starts_s, counts_s, max_deg, mode="max",
                     E=E)
    h2 = _tc_combine(agg2, h1, W_l2, W_r2, b2r, cnt_col, mode="max")
    m3 = _lstm_pass(h2, src_slot, starts_s, counts_s, max_deg, W_ih3, W_hh3,
                    b3s, E=E)
    h3 = _tc_combine(m3, h2, W_l3, W_r3, b3r, cnt_col, mode="plain")
    hout = _tc_jk(h1, h2, h3, Wf_ih, Wf_hh, bfr, Wb_ih, Wb_hh, bbr, watt, H=H)
    hor = _sc_permute(hout, slot_of.reshape(_NW, RP, _CHK)).reshape(NP, D)
    h = hor[:N]
    return (h, h)


# speedup vs baseline: 14.7497x; 1.0491x over previous
import functools

import jax
import jax.numpy as jnp
from jax import lax
from jax.experimental import pallas as pl
from jax.experimental.pallas import tpu as pltpu
from jax.experimental.pallas import tpu_sc as plsc

_NC = 2
_NS = 16
_NW = _NC * _NS
_CHK = 80
_RPW = 4
C = 16


def _sc_gather_chunk_w(table, src_slot, starts4, t0v, *, W):
    NP = table.shape[0]
    D = table.shape[1]
    E = src_slot.shape[0]
    NROW = NP // _CHK

    mesh = plsc.VectorSubcoreMesh(core_axis_name="c", subcore_axis_name="s")

    @functools.partial(
        pl.kernel,
        mesh=mesh,
        out_type=jax.ShapeDtypeStruct((C, NROW, _CHK, D), jnp.float32),
        scratch_types=[
            pltpu.VMEM((_RPW, _CHK), jnp.int32),
            pltpu.VMEM((16,), jnp.int32),
            pltpu.VMEM((C, _RPW, _CHK), jnp.int32),
            pltpu.VMEM((C, _RPW, _CHK), jnp.int32),
            pltpu.VMEM((W, _CHK, D), jnp.float32),
            pltpu.VMEM((W, _CHK, D), jnp.float32),
            pltpu.SemaphoreType.DMA,
            pltpu.SemaphoreType.DMA,
            pltpu.SemaphoreType.DMA,
        ],
    )
    def k(table_hbm, srcs_hbm, starts_hbm, t0_hbm, out_hbm, st_v, t0_v,
          idx_v, sid_v, r0, r1, sem_s, sem_r, sem_o):
        wid = lax.axis_index("s") * _NC + lax.axis_index("c")
        row0 = wid * _RPW
        bufs = (r0, r1)
        pltpu.sync_copy(t0_hbm, t0_v)
        pltpu.sync_copy(starts_hbm.at[wid], st_v)
        tv = t0_v[...]
        for c in range(C):
            for j in range(W):
                for v in range(_CHK // 16):
                    sl = st_v[j, pl.ds(v * 16, 16)]
                    idx_v[c, j, pl.ds(v * 16, 16)] = jnp.minimum(
                        sl + tv + c, E - 1)
        sid_h = [[pltpu.async_copy(srcs_hbm.at[idx_v.at[c, j]],
                                   sid_v.at[c, j], sem_s)
                  for j in range(W)] for c in range(C)]

        def fire_rows(c):
            for h in sid_h[c]:
                h.wait()
            buf = bufs[c % 2]
            return [pltpu.async_copy(table_hbm.at[sid_v.at[c, j]],
                                     buf.at[j], sem_r)
                    for j in range(W)]

        row_h = fire_rows(0)
        out_h = [None] * C
        for c in range(C):
            for h in row_h:
                h.wait()
            out_h[c] = pltpu.async_copy(
                bufs[c % 2], out_hbm.at[c, pl.ds(row0, W)], sem_o)
            if c + 1 < C:
                if c >= 1:
                    out_h[c - 1].wait()
                row_h = fire_rows(c + 1)
        if C >= 2:
            out_h[C - 2].wait()
        out_h[C - 1].wait()

    return k(table, src_slot, starts4, t0v)


def _sc_gather_chunk(table, src_slot, starts4, t0v, nch0):
    branches = [functools.partial(_sc_gather_chunk_w, W=w)
                for w in range(1, _RPW + 1)]
    sel = jnp.clip(nch0, 1, _RPW) - 1
    return lax.switch(sel, branches, table, src_slot, starts4, t0v)


def _sc_permute(table, perm3):
    NP = table.shape[0]
    D = table.shape[1]
    NROW = NP // _CHK
    RP = NROW // _NW

    mesh = plsc.VectorSubcoreMesh(core_axis_name="c", subcore_axis_name="s")

    @functools.partial(
        pl.kernel,
        mesh=mesh,
        out_type=jax.ShapeDtypeStruct((NROW, _CHK, D), jnp.float32),
        scratch_types=[
            pltpu.VMEM((RP, _CHK), jnp.int32),
            pltpu.VMEM((RP, _CHK, D), jnp.float32),
            pltpu.SemaphoreType.DMA,
        ],
    )
    def k(table_hbm, perm_hbm, out_hbm, idx_v, rows_v, sem):
        wid = lax.axis_index("s") * _NC + lax.axis_index("c")
        row0 = wid * RP
        pltpu.sync_copy(perm_hbm.at[wid], idx_v)
        hs = [pltpu.async_copy(table_hbm.at[idx_v.at[j]], rows_v.at[j], sem)
              for j in range(RP)]
        for h in hs:
            h.wait()
        pltpu.sync_copy(rows_v, out_hbm.at[pl.ds(row0, RP)])

    return k(table, perm3)


def _tc_reduce(scal, Xc, cnt, acc, *, mode):
    NP, D = acc.shape
    blk = 1024

    def body(scal_ref, xc_ref, cnt_ref, acc_ref, out_ref):
        a = acc_ref[...]
        t0 = scal_ref[0]
        cnt = cnt_ref[...]
        for s in range(C):
            x = xc_ref[s]
            valid = cnt > t0 + s
            if mode == "sum":
                a = a + jnp.where(valid, x, 0.0)
            else:
                a = jnp.maximum(a, jnp.where(valid, x, -jnp.inf))
        out_ref[...] = a

    return pl.pallas_call(
        body,
        grid=(NP // blk,),
        in_specs=[
            pl.BlockSpec(memory_space=pltpu.SMEM),
            pl.BlockSpec((C, blk, D), lambda i: (0, i, 0)),
            pl.BlockSpec((blk, 1), lambda i: (i, 0)),
            pl.BlockSpec((blk, D), lambda i: (i, 0)),
        ],
        out_specs=pl.BlockSpec((blk, D), lambda i: (i, 0)),
        out_shape=jax.ShapeDtypeStruct((NP, D), jnp.float32),
        input_output_aliases={3: 0},
    )(scal, Xc, cnt, acc)


def _tc_lstm_chunk(scal, Xc, cnt, h, c, W_ih, W_hh, bias):
    NP, D = h.shape
    blk = 512

    def body(md_ref, xc_ref, cnt_ref, h_ref, c_ref, wi_ref, wh_ref, b_ref,
             ho_ref, co_ref):
        hh = h_ref[...]
        cc = c_ref[...]
        cnt = cnt_ref[...]
        wi = wi_ref[...]
        wh = wh_ref[...]
        b = b_ref[...]
        t0 = md_ref[0]
        md = md_ref[1]
        for s in range(C):
            x = jnp.where(cnt > t0 + s, xc_ref[s], 0.0)
            g = (jnp.dot(x, wi, preferred_element_type=jnp.float32)
                 + jnp.dot(hh, wh, preferred_element_type=jnp.float32) + b)
            gi = jax.nn.sigmoid(g[:, 0 * D:1 * D])
            gf = jax.nn.sigmoid(g[:, 1 * D:2 * D])
            gg = jnp.tanh(g[:, 2 * D:3 * D])
            go = jax.nn.sigmoid(g[:, 3 * D:4 * D])
            cn = gf * cc + gi * gg
            hn = go * jnp.tanh(cn)
            upd = t0 + s < md
            hh = jnp.where(upd, hn, hh)
            cc = jnp.where(upd, cn, cc)
        ho_ref[...] = hh
        co_ref[...] = cc

    return pl.pallas_call(
        body,
        grid=(NP // blk,),
        in_specs=[
            pl.BlockSpec(memory_space=pltpu.SMEM),
            pl.BlockSpec((C, blk, D), lambda i: (0, i, 0)),
            pl.BlockSpec((blk, 1), lambda i: (i, 0)),
            pl.BlockSpec((blk, D), lambda i: (i, 0)),
            pl.BlockSpec((blk, D), lambda i: (i, 0)),
            pl.BlockSpec((D, 4 * D), lambda i: (0, 0)),
            pl.BlockSpec((D, 4 * D), lambda i: (0, 0)),
            pl.BlockSpec((1, 4 * D), lambda i: (0, 0)),
        ],
        out_specs=[
            pl.BlockSpec((blk, D), lambda i: (i, 0)),
            pl.BlockSpec((blk, D), lambda i: (i, 0)),
        ],
        out_shape=[
            jax.ShapeDtypeStruct((NP, D), jnp.float32),
            jax.ShapeDtypeStruct((NP, D), jnp.float32),
        ],
        input_output_aliases={3: 0, 4: 1},
    )(scal, Xc, cnt, h, c, W_ih, W_hh, bias)


def _tc_combine(agg, hprev, Wl, Wr, b, cnt, *, mode):
    NP, D = agg.shape
    blk = 512

    def body(agg_ref, hp_ref, wl_ref, wr_ref, b_ref, cnt_ref, out_ref):
        a = agg_ref[...]
        cntf = cnt_ref[...].astype(jnp.float32)
        if mode == "mean":
            a = a / jnp.maximum(cntf, 1.0)
        elif mode == "max":
            a = jnp.where(cntf > 0.0, a, 0.0)
        o = (jnp.dot(a, wl_ref[...], preferred_element_type=jnp.float32)
             + jnp.dot(hp_ref[...], wr_ref[...],
                       preferred_element_type=jnp.float32)
             + b_ref[...])
        if mode != "plain":
            o = jnp.maximum(o, 0.0)
        out_ref[...] = o

    return pl.pallas_call(
        body,
        grid=(NP // blk,),
        in_specs=[
            pl.BlockSpec((blk, D), lambda i: (i, 0)),
            pl.BlockSpec((blk, D), lambda i: (i, 0)),
            pl.BlockSpec((D, D), lambda i: (0, 0)),
            pl.BlockSpec((D, D), lambda i: (0, 0)),
            pl.BlockSpec((1, D), lambda i: (0, 0)),
            pl.BlockSpec((blk, 1), lambda i: (i, 0)),
        ],
        out_specs=pl.BlockSpec((blk, D), lambda i: (i, 0)),
        out_shape=jax.ShapeDtypeStruct((NP, D), jnp.float32),
    )(agg, hprev, Wl, Wr, b, cnt)


def _tc_jk(h1, h2, h3, Wf_ih, Wf_hh, bf, Wb_ih, Wb_hh, bb, watt, *, H):
    NP, D = h1.shape
    blk = 512

    def body(h1_ref, h2_ref, h3_ref, wfi_ref, wfh_ref, bf_ref,
             wbi_ref, wbh_ref, bb_ref, wa_ref, out_ref):
        x1 = h1_ref[...]
        x2 = h2_ref[...]
        x3 = h3_ref[...]
        seq = (x1, x2, x3)

        def cell(x, h, c, wi, wh, b):
            g = (jnp.dot(x, wi, preferred_element_type=jnp.float32)
                 + jnp.dot(h, wh, preferred_element_type=jnp.float32) + b)
            gi = jax.nn.sigmoid(g[:, 0 * H:1 * H])
            gf = jax.nn.sigmoid(g[:, 1 * H:2 * H])
            gg = jnp.tanh(g[:, 2 * H:3 * H])
            go = jax.nn.sigmoid(g[:, 3 * H:4 * H])
            c2 = gf * c + gi * gg
            return go * jnp.tanh(c2), c2

        wfi = wfi_ref[...]
        wfh = wfh_ref[...]
        bfv = bf_ref[...]
        wbi = wbi_ref[...]
        wbh = wbh_ref[...]
        bbv = bb_ref[...]
        z = jnp.zeros((x1.shape[0], H), jnp.float32)
        hf, cf = z, z
        hs_f = []
        for t in range(3):
            hf, cf = cell(seq[t], hf, cf, wfi, wfh, bfv)
            hs_f.append(hf)
        hb, cb = z, z
        hs_b = [None, None, None]
        for k in range(3):
            t = 2 - k
            hb, cb = cell(seq[t], hb, cb, wbi, wbh, bbv)
            hs_b[t] = hb
        wa = wa_ref[...]
        wa_f = wa[:, :H]
        wa_b = wa[:, H:]
        atts = []
        for t in range(3):
            att = (jnp.sum(hs_f[t] * wa_f, axis=1, keepdims=True)
                   + jnp.sum(hs_b[t] * wa_b, axis=1, keepdims=True))
            atts.append(att)
        m = jnp.maximum(atts[0], jnp.maximum(atts[1], atts[2]))
        e0 = jnp.exp(atts[0] - m)
        e1 = jnp.exp(atts[1] - m)
        e2 = jnp.exp(atts[2] - m)
        z_sum = e0 + e1 + e2
        out_ref[...] = (e0 * x1 + e1 * x2 + e2 * x3) / z_sum

    return pl.pallas_call(
        body,
        grid=(NP // blk,),
        in_specs=[
            pl.BlockSpec((blk, D), lambda i: (i, 0)),
            pl.BlockSpec((blk, D), lambda i: (i, 0)),
            pl.BlockSpec((blk, D), lambda i: (i, 0)),
            pl.BlockSpec((D, 4 * H), lambda i: (0, 0)),
            pl.BlockSpec((H, 4 * H), lambda i: (0, 0)),
            pl.BlockSpec((1, 4 * H), lambda i: (0, 0)),
            pl.BlockSpec((D, 4 * H), lambda i: (0, 0)),
            pl.BlockSpec((H, 4 * H), lambda i: (0, 0)),
            pl.BlockSpec((1, 4 * H), lambda i: (0, 0)),
            pl.BlockSpec((1, 2 * H), lambda i: (0, 0)),
        ],
        out_specs=pl.BlockSpec((blk, D), lambda i: (i, 0)),
        out_shape=jax.ShapeDtypeStruct((NP, D), jnp.float32),
    )(h1, h2, h3, Wf_ih, Wf_hh, bf, Wb_ih, Wb_hh, bb, watt)


def _chunk_meta(counts_s, max_deg, t0):
    act = jnp.sum(counts_s > t0)
    per_w = (act + _NW - 1) // _NW
    nch0 = ((per_w + _CHK - 1) // _CHK).astype(jnp.int32)
    t0v = jnp.full((16,), t0, jnp.int32)
    scal = jnp.stack([t0, max_deg]).astype(jnp.int32)
    return nch0, t0v, scal


def _agg_pass(table, src_slot, starts4, counts_s, cnt_col, max_deg, *, mode):
    NP, D = table.shape
    if mode == "sum":
        init = jnp.zeros((NP, D), jnp.float32)
    else:
        init = jnp.full((NP, D), -jnp.inf, jnp.float32)
    K = (max_deg + C - 1) // C

    def cond(st):
        return st[0] < K

    def body(st):
        i, acc = st
        t0 = i * C
        nch0, t0v, scal = _chunk_meta(counts_s, max_deg, t0)
        Xc = _sc_gather_chunk(table, src_slot, starts4, t0v,
                              nch0).reshape(C, NP, D)
        acc = _tc_reduce(scal, Xc, cnt_col, acc, mode=mode)
        return (i + jnp.int32(1), acc)

    _, acc = lax.while_loop(cond, body, (jnp.int32(0), init))
    return acc


def _lstm_pass(table, src_slot, starts4, counts_s, cnt_col, max_deg,
               W_ih, W_hh, bias):
    NP, D = table.shape
    K = (max_deg + C - 1) // C
    h0 = jnp.zeros((NP, D), jnp.float32)
    c0 = jnp.zeros((NP, D), jnp.float32)

    def cond(st):
        return st[0] < K

    def body(st):
        i, h, c = st
        t0 = i * C
        nch0, t0v, scal = _chunk_meta(counts_s, max_deg, t0)
        Xc = _sc_gather_chunk(table, src_slot, starts4, t0v,
                              nch0).reshape(C, NP, D)
        h, c = _tc_lstm_chunk(scal, Xc, cnt_col, h, c, W_ih, W_hh, bias)
        return (i + jnp.int32(1), h, c)

    _, h, _ = lax.while_loop(cond, body, (jnp.int32(0), h0, c0))
    return h


def kernel(x, edge_index, W_l1, W_r1, b1, W_l2, W_r2, b2, W_ih3, W_hh3, b_ih3,
           b_hh3, W_l3, W_r3, b3, Wf_ih, Wf_hh, bf_ih, bf_hh, Wb_ih, Wb_hh,
           bb_ih, bb_hh, W_att, b_att):
    N, D = x.shape
    E = edge_index.shape[1]
    H = Wf_hh.shape[0]
    GR = _NW * _RPW * _CHK
    NP = ((N + GR - 1) // GR) * GR

    src = edge_index[0].astype(jnp.int32)
    dst = edge_index[1].astype(jnp.int32)
    perm = jnp.argsort(dst, stable=True)
    src_s = src[perm]
    counts = jnp.bincount(dst, length=N).astype(jnp.int32)
    starts = jnp.concatenate(
        [jnp.zeros((1,), jnp.int32), jnp.cumsum(counts)[:-1].astype(jnp.int32)])
    max_deg = jnp.max(counts)

    x_p = jnp.zeros((NP, D), jnp.float32).at[:N].set(x)
    counts_p = jnp.zeros((NP,), jnp.int32).at[:N].set(counts)
    starts_p = jnp.zeros((NP,), jnp.int32).at[:N].set(starts)

    SPW = NP // _NW
    RP = SPW // _CHK
    nperm = jnp.argsort(-counts_p).astype(jnp.int32)
    slotperm = nperm.reshape(SPW, _NW).transpose(1, 0).reshape(NP)
    slot_of = jnp.zeros((NP,), jnp.int32).at[slotperm].set(
        jnp.arange(NP, dtype=jnp.int32))
    counts_s = counts_p[slotperm]
    starts_s = starts_p[slotperm]
    src_slot = slot_of[src_s]
    x_s = _sc_permute(x_p, slotperm.reshape(_NW, RP, _CHK)).reshape(NP, D)
    cnt_col = counts_s[:, None]
    starts4 = starts_s.reshape(_NW, _RPW, _CHK)

    b1r = b1.reshape(1, D)
    b2r = b2.reshape(1, D)
    b3r = b3.reshape(1, D)
    b3s = (b_ih3 + b_hh3).reshape(1, 4 * D)
    bfr = (bf_ih + bf_hh).reshape(1, 4 * H)
    bbr = (bb_ih + bb_hh).reshape(1, 4 * H)
    watt = W_att.reshape(1, 2 * H)

    agg1 = _agg_pass(x_s, src_slot, starts4, counts_s, cnt_col, max_deg,
                     mode="sum")
    h1 = _tc_combine(agg1, x_s, W_l1, W_r1, b1r, cnt_col, mode="mean")
    agg2 = _agg_pass(h1, src_slot, starts4, counts_s, cnt_col, max_deg,
                     mode="max")
    h2 = _tc_combine(agg2, h1, W_l2, W_r2, b2r, cnt_col, mode="max")
    m3 = _lstm_pass(h2, src_slot, starts4, counts_s, cnt_col, max_deg,
                    W_ih3, W_hh3, b3s)
    h3 = _tc_combine(m3, h2, W_l3, W_r3, b3r, cnt_col, mode="plain")
    hout = _tc_jk(h1, h2, h3, Wf_ih, Wf_hh, bfr, Wb_ih, Wb_hh, bbr, watt, H=H)
    hor = _sc_permute(hout, slot_of.reshape(_NW, RP, _CHK)).reshape(NP, D)
    h = hor[:N]
    return (h, h)
